# baseline (speedup 1.0000x reference)
import jax
import jax.numpy as jnp
import numpy as np
from jax import lax
from jax.experimental import pallas as pl
from jax.experimental.pallas import tpu as pltpu
from jax.experimental.pallas import tpu_sc as plsc

_MAX_REL = 128
_SQ = 2048
_TQ = 256
_TK = 256
_L = 512
_ND = 15
_SHIFT = _SQ - _MAX_REL - 1

_NT = _SQ // _TQ

_BAND = [(q, k) for q in range(_NT) for k in range(_NT) if abs(k - q) <= 1]
_OFF = [(q, k) for q in range(_NT) for k in range(_NT) if abs(k - q) > 1]
_QS_A = np.array([q for q, _ in _OFF], np.int32)
_KS_A = np.array([k for _, k in _OFF], np.int32)
_QS_B = np.array([q for q, _ in _BAND], np.int32)
_KS_B = np.array([k for _, k in _BAND], np.int32)


def _sc_windows(t_hbm, f_hbm, t_v, row_v):
    wid = lax.axis_index("s") * 2 + lax.axis_index("c")

    @pl.when(wid < _ND)
    def _():
        pltpu.sync_copy(t_hbm, t_v)
        lane = lax.iota(jnp.int32, 16)

        def chunk(c, carry):
            idx = jnp.clip(256 * wid + 16 * c + lane - _SHIFT, 0, 2 * _MAX_REL)
            row_v[pl.ds(16 * c, 16)] = plsc.load_gather(t_v, [idx])
            return carry

        lax.fori_loop(0, _L // 16, chunk, 0)
        pltpu.sync_copy(row_v, f_hbm.at[wid, 0])


def _build_windows(relative_biases):
    mesh = plsc.VectorSubcoreMesh(core_axis_name="c", subcore_axis_name="s")
    cp = pltpu.CompilerParams(needs_layout_passes=False)
    return pl.kernel(
        _sc_windows,
        mesh=mesh,
        compiler_params=cp,
        out_type=jax.ShapeDtypeStruct((_ND, 1, _L), jnp.float32),
        scratch_types=[
            pltpu.VMEM((257,), jnp.float32),
            pltpu.VMEM((_L,), jnp.float32),
        ],
    )(relative_biases)


def _body_a(qs_ref, ks_ref, t_ref, x_ref, o_ref):
    t = pl.program_id(0)
    d = ks_ref[t] - qs_ref[t] + 7
    bias = jnp.where(d <= 5, t_ref[0], t_ref[2 * _MAX_REL])
    o_ref[...] = x_ref[...] + bias


def _body_b(qs_ref, ks_ref, f_ref, x_ref, oa_ref, o_ref):
    del oa_ref
    f = f_ref[0, 0, :]
    fb = jnp.broadcast_to(f[None, :], (_TQ, _L))
    bias = pltpu.roll(fb, _L - _TQ + 1, axis=1, stride=1, stride_axis=0)
    o_ref[...] = x_ref[...] + bias[None, :, :_TK]


def kernel(inputs, relative_biases):
    f_all = _build_windows(relative_biases)
    b = inputs.shape[0]
    oshape = jax.ShapeDtypeStruct(inputs.shape, inputs.dtype)

    x_spec = pl.BlockSpec((b, _TQ, _TK), lambda t, qs, ks: (0, qs[t], ks[t]))

    out_a = pl.pallas_call(
        _body_a,
        grid_spec=pltpu.PrefetchScalarGridSpec(
            num_scalar_prefetch=2,
            grid=(len(_OFF),),
            in_specs=[
                pl.BlockSpec((2 * _MAX_REL + 1,), lambda t, qs, ks: (0,)),
                x_spec,
            ],
            out_specs=x_spec,
        ),
        out_shape=oshape,
    )(jnp.asarray(_QS_A), jnp.asarray(_KS_A), relative_biases, inputs)

    out = pl.pallas_call(
        _body_b,
        grid_spec=pltpu.PrefetchScalarGridSpec(
            num_scalar_prefetch=2,
            grid=(len(_BAND),),
            in_specs=[
                pl.BlockSpec((1, 1, _L), lambda t, qs, ks: (ks[t] - qs[t] + 7, 0, 0)),
                x_spec,
                pl.BlockSpec(memory_space=pl.ANY),
            ],
            out_specs=x_spec,
        ),
        out_shape=oshape,
        input_output_aliases={4: 0},
    )(jnp.asarray(_QS_B), jnp.asarray(_KS_B), f_all, inputs, out_a)
    return out


# speedup vs baseline: 1.0048x; 1.0048x over previous
import jax
import jax.numpy as jnp
import numpy as np
from jax import lax
from jax.experimental import pallas as pl
from jax.experimental.pallas import tpu as pltpu
from jax.experimental.pallas import tpu_sc as plsc

_MAX_REL = 128
_SQ = 2048
_TQ = 256
_TK = 256
_L = 512
_ND = 15
_SHIFT = _SQ - _MAX_REL - 1

_NT = _SQ // _TQ

_BAND = [(q, k) for q in range(_NT) for k in range(_NT) if abs(k - q) <= 1]
_OFF = [(q, k) for q in range(_NT) for k in range(_NT) if abs(k - q) > 1]
_QS_A = np.array([q for q, _ in _OFF], np.int32)
_KS_A = np.array([k for _, k in _OFF], np.int32)
_QS_B = np.array([q for q, _ in _BAND], np.int32)
_KS_B = np.array([k for _, k in _BAND], np.int32)


def _sc_windows(t_hbm, f_hbm, t_v, row_v):
    wid = lax.axis_index("s") * 2 + lax.axis_index("c")

    @pl.when(wid < _ND)
    def _():
        pltpu.sync_copy(t_hbm, t_v)
        lane = lax.iota(jnp.int32, 16)

        def chunk(c, carry):
            idx = jnp.clip(256 * wid + 16 * c + lane - _SHIFT, 0, 2 * _MAX_REL)
            row_v[pl.ds(16 * c, 16)] = plsc.load_gather(t_v, [idx])
            return carry

        lax.fori_loop(0, _L // 16, chunk, 0)
        pltpu.sync_copy(row_v, f_hbm.at[wid, 0])


def _build_windows(relative_biases):
    mesh = plsc.VectorSubcoreMesh(core_axis_name="c", subcore_axis_name="s")
    cp = pltpu.CompilerParams(needs_layout_passes=False)
    return pl.kernel(
        _sc_windows,
        mesh=mesh,
        compiler_params=cp,
        out_type=jax.ShapeDtypeStruct((_ND, 1, _L), jnp.float32),
        scratch_types=[
            pltpu.VMEM((257,), jnp.float32),
            pltpu.VMEM((_L,), jnp.float32),
        ],
    )(relative_biases)


def _body_a(qs_ref, ks_ref, t_ref, x_ref, o_ref):
    t = pl.program_id(0)
    d = ks_ref[t] - qs_ref[t] + 7
    bias = jnp.where(d <= 5, t_ref[0], t_ref[2 * _MAX_REL])
    o_ref[...] = x_ref[...] + bias


def _body_b(f_ref, x_ref, oa_ref, o_ref):
    del oa_ref
    f = f_ref[0, 0, :]
    fb = jnp.broadcast_to(f[None, :], (_TQ, _L))
    bias = pltpu.roll(fb, _L - _TQ + 1, axis=1, stride=1, stride_axis=0)
    o_ref[...] = x_ref[...] + bias[None, :, :_TK]


def _b_qi(t):
    return (t + 1) // 3


def _b_ki(t):
    return _b_qi(t) + (t + 1) % 3 - 1


def kernel(inputs, relative_biases):
    f_all = _build_windows(relative_biases)
    b = inputs.shape[0]
    oshape = jax.ShapeDtypeStruct(inputs.shape, inputs.dtype)

    x_spec = pl.BlockSpec((b, _TQ, _TK), lambda t, qs, ks: (0, qs[t], ks[t]))

    out_a = pl.pallas_call(
        _body_a,
        grid_spec=pltpu.PrefetchScalarGridSpec(
            num_scalar_prefetch=2,
            grid=(len(_OFF),),
            in_specs=[
                pl.BlockSpec((2 * _MAX_REL + 1,), lambda t, qs, ks: (0,)),
                x_spec,
            ],
            out_specs=x_spec,
        ),
        out_shape=oshape,
    )(jnp.asarray(_QS_A), jnp.asarray(_KS_A), relative_biases, inputs)

    xb_spec = pl.BlockSpec((b, _TQ, _TK), lambda t: (0, _b_qi(t), _b_ki(t)))
    out = pl.pallas_call(
        _body_b,
        grid=(len(_BAND),),
        in_specs=[
            pl.BlockSpec((1, 1, _L), lambda t: ((t + 1) % 3 + 6, 0, 0)),
            xb_spec,
            pl.BlockSpec(memory_space=pl.ANY),
        ],
        out_specs=xb_spec,
        out_shape=oshape,
        input_output_aliases={2: 0},
    )(f_all, inputs, out_a)
    return out
